# baseline (speedup 1.0000x reference)
import functools

import numpy as np

import jax
import jax.numpy as jnp
from jax import lax
from jax.experimental import pallas as pl
from jax.experimental.pallas import tpu as pltpu
from jax.experimental.pallas import tpu_sc as plsc

IN_FEATURES = 512
NUM_CODEBOOKS = 2
NUM_CODEWORDS = 320
NCOL = NUM_CODEBOOKS * NUM_CODEWORDS
CODEWORD_DIM = 128
ROWS = 4 * 1024
BLK = 1024
GRID = ROWS // BLK
IDR = NUM_CODEBOOKS * BLK // 128

NC, NS = 2, 16
NW = NC * NS
W_PER_G = NW // GRID
T_PER_CB = BLK // 256


def _tc_body(
    x_ref, w_ref, b_ref, iota_ref, aux_ref, ids_ref, perp_ref, counts_ref, idx_s
):
    pid = pl.program_id(0)

    @pl.when(pid == 0)
    def _init():
        counts_ref[...] = jnp.zeros_like(counts_ref)

    logits = (
        jnp.dot(x_ref[0], w_ref[...], preferred_element_type=jnp.float32)
        + b_ref[...].reshape(1, NCOL)
    )
    iota_row = iota_ref[...]
    big = jnp.int32(2**30)
    neg = jnp.float32(-1e30)
    mask0_row = iota_row < NUM_CODEWORDS
    tie_flags = []
    for n in range(NUM_CODEBOOKS):
        mrow = mask0_row if n == 0 else jnp.logical_not(mask0_row)
        lm = jnp.where(jnp.broadcast_to(mrow, (BLK, NCOL)), logits, neg)
        m = jnp.max(lm, axis=1, keepdims=True)
        eqf = (lm == m).astype(jnp.float32)
        hlc = lax.dot_general(
            eqf,
            aux_ref[...],
            (((1,), (0,)), ((), ())),
            preferred_element_type=jnp.float32,
        )
        idx_f = hlc[:, 0:1] * 128.0 + hlc[:, 1:2]
        idx_s[:, n : n + 1] = idx_f.astype(jnp.int32)
        counts_ref[...] += lax.dot_general(
            jnp.ones((1, BLK), jnp.float32),
            eqf,
            (((1,), (0,)), ((), ())),
            preferred_element_type=jnp.float32,
        )
        tie_flags.append(jnp.max(hlc[:, 2:3]))

    @pl.when(jnp.maximum(tie_flags[0], tie_flags[1]) > 1.5)
    def _tie_fixup():
        iota_b = jnp.broadcast_to(iota_row, (BLK, NCOL))
        for n in range(NUM_CODEBOOKS):
            mrow = mask0_row if n == 0 else jnp.logical_not(mask0_row)
            lm = jnp.where(jnp.broadcast_to(mrow, (BLK, NCOL)), logits, neg)
            m = jnp.max(lm, axis=1, keepdims=True)
            eqf = (lm == m).astype(jnp.float32)
            cand = jnp.where(lm == m, iota_b, big)
            idx = jnp.min(cand, axis=1, keepdims=True)
            idx_s[:, n : n + 1] = idx
            onehot = (iota_b == idx).astype(jnp.float32)
            counts_ref[...] += lax.dot_general(
                jnp.ones((1, BLK), jnp.float32),
                onehot - eqf,
                (((1,), (0,)), ((), ())),
                preferred_element_type=jnp.float32,
            )

    idx_lane = idx_s[...].T
    pieces = [
        idx_lane[n : n + 1, 128 * k : 128 * (k + 1)]
        for n in range(NUM_CODEBOOKS)
        for k in range(BLK // 128)
    ]
    ids_ref[...] = jnp.concatenate(pieces, axis=0)

    @pl.when(pid == GRID - 1)
    def _fin():
        p = counts_ref[...] * (1.0 / ROWS)
        plogp = p * jnp.log(p + 1e-7)
        e0 = jnp.sum(plogp[:, :NUM_CODEWORDS])
        e1 = jnp.sum(plogp[:, NUM_CODEWORDS:])
        perp_ref[...] = jnp.broadcast_to(jnp.exp(-e0) + jnp.exp(-e1), (1, 1))


def _tc_stage(x, w, b_vec, iota_row, aux_cols):
    return pl.pallas_call(
        _tc_body,
        grid=(GRID,),
        in_specs=[
            pl.BlockSpec(
                (1, BLK, IN_FEATURES),
                lambda i: (i // (1024 // BLK), i % (1024 // BLK), 0),
            ),
            pl.BlockSpec((IN_FEATURES, NCOL), lambda i: (0, 0)),
            pl.BlockSpec((NCOL,), lambda i: (0,)),
            pl.BlockSpec((1, NCOL), lambda i: (0, 0)),
            pl.BlockSpec((NCOL, 3), lambda i: (0, 0)),
        ],
        out_specs=[
            pl.BlockSpec((IDR, 128), lambda i: (i, 0)),
            pl.BlockSpec((1, 1), lambda i: (0, 0)),
        ],
        out_shape=[
            jax.ShapeDtypeStruct((IDR * GRID, 128), jnp.int32),
            jax.ShapeDtypeStruct((1, 1), jnp.float32),
        ],
        scratch_shapes=[
            pltpu.VMEM((1, NCOL), jnp.float32),
            pltpu.VMEM((BLK, NUM_CODEBOOKS), jnp.int32),
        ],
    )(x, w, b_vec, iota_row, aux_cols)


@functools.lru_cache(maxsize=1)
def _make_sc_gather():
    @functools.partial(
        pl.kernel,
        mesh=plsc.VectorSubcoreMesh(core_axis_name="c", subcore_axis_name="s"),
        out_type=jax.ShapeDtypeStruct((ROWS, NUM_CODEBOOKS * CODEWORD_DIM), jnp.float32),
        scratch_types=[
            pltpu.VMEM((2, 128), jnp.int32),
            pltpu.VMEM((256, CODEWORD_DIM), jnp.float32),
            pltpu.SemaphoreType.DMA,
            pltpu.SemaphoreType.DMA,
        ],
    )
    def _sc_gather(table_hbm, idx_hbm, out_hbm, idx_v, rows_v, sem, wsem):
        wid = lax.axis_index("s") * NC + lax.axis_index("c")
        g = wid // W_PER_G
        q = wid % W_PER_G
        n = q // T_PER_CB
        t = q % T_PER_CB
        pltpu.sync_copy(
            idx_hbm.at[pl.ds(IDR * g + (IDR // 2) * n + 2 * t, 2)], idx_v
        )
        row0 = BLK * g + 256 * t
        col = 128 * n
        gathers = [
            pltpu.async_copy(
                table_hbm.at[idx_v.at[j]], rows_v.at[pl.ds(j * 128, 128)], sem
            )
            for j in range(2)
        ]
        writes = []
        for j in range(2):
            gathers[j].wait()
            writes.append(
                pltpu.async_copy(
                    rows_v.at[pl.ds(j * 128, 128)],
                    out_hbm.at[pl.ds(row0 + j * 128, 128), pl.ds(col, 128)],
                    wsem,
                )
            )
        for w in writes:
            w.wait()

    return _sc_gather


_IOTA_ROW = np.arange(NCOL, dtype=np.int32).reshape(1, NCOL)
_AUX_COLS = np.stack(
    [
        np.arange(NCOL) // 128,
        np.arange(NCOL) % 128,
        np.ones(NCOL),
    ],
    axis=1,
).astype(np.float32)


def kernel(x, codebooks, W, b):
    bsz, nf, _ = x.shape
    ids, perp = _tc_stage(
        x, W, b, jnp.asarray(_IOTA_ROW), jnp.asarray(_AUX_COLS)
    )
    table = codebooks.reshape(NCOL, CODEWORD_DIM)
    rows = _make_sc_gather()(table, ids)
    quantized = rows.reshape(bsz, nf, NUM_CODEBOOKS * CODEWORD_DIM)
    return quantized, perp.reshape(())


# speedup vs baseline: 1.0367x; 1.0367x over previous
import functools

import numpy as np

import jax
import jax.numpy as jnp
from jax import lax
from jax.experimental import pallas as pl
from jax.experimental.pallas import tpu as pltpu
from jax.experimental.pallas import tpu_sc as plsc

IN_FEATURES = 512
NUM_CODEBOOKS = 2
NUM_CODEWORDS = 320
NCOL = NUM_CODEBOOKS * NUM_CODEWORDS
CODEWORD_DIM = 128
ROWS = 4 * 1024
BLK = 512
GRID = ROWS // BLK
IDR = NUM_CODEBOOKS * BLK // 128

NC, NS = 2, 16
NW = NC * NS
W_PER_G = NW // GRID
T_PER_CB = BLK // 256


def _tc_body(x_ref, w_ref, b_ref, iota_ref, ids_ref, perp_ref, counts_ref):
    pid = pl.program_id(0)

    @pl.when(pid == 0)
    def _init():
        counts_ref[...] = jnp.zeros_like(counts_ref)

    logits = (
        jnp.dot(x_ref[0], w_ref[...], preferred_element_type=jnp.float32)
        + b_ref[...].reshape(1, NCOL)
    )
    iota_row = iota_ref[...]
    big = jnp.int32(2**30)
    neg = jnp.float32(-1e30)
    mask0_row = iota_row < NUM_CODEWORDS
    iota_b = jnp.broadcast_to(iota_row, (BLK, NCOL))
    idxs = []
    for n in range(NUM_CODEBOOKS):
        mrow = mask0_row if n == 0 else jnp.logical_not(mask0_row)
        lm = jnp.where(jnp.broadcast_to(mrow, (BLK, NCOL)), logits, neg)
        m = jnp.max(lm, axis=1, keepdims=True)
        cand = jnp.where(lm == m, iota_b, big)
        idxs.append(jnp.min(cand, axis=1, keepdims=True))
    onehot = ((iota_b == idxs[0]) | (iota_b == idxs[1])).astype(jnp.float32)
    counts_ref[...] += lax.dot_general(
        jnp.ones((1, BLK), jnp.float32),
        onehot,
        (((1,), (0,)), ((), ())),
        preferred_element_type=jnp.float32,
    )
    idx_lane = jnp.concatenate(idxs, axis=1).T
    pieces = [
        idx_lane[n : n + 1, 128 * k : 128 * (k + 1)]
        for n in range(NUM_CODEBOOKS)
        for k in range(BLK // 128)
    ]
    ids_ref[...] = jnp.concatenate(pieces, axis=0)

    @pl.when(pid == GRID - 1)
    def _fin():
        p = counts_ref[...] * (1.0 / ROWS)
        plogp = p * jnp.log(p + 1e-7)
        e0 = jnp.sum(plogp[:, :NUM_CODEWORDS])
        e1 = jnp.sum(plogp[:, NUM_CODEWORDS:])
        perp_ref[...] = jnp.broadcast_to(jnp.exp(-e0) + jnp.exp(-e1), (1, 1))


def _tc_stage(x, w, b_vec, iota_row):
    return pl.pallas_call(
        _tc_body,
        grid=(GRID,),
        in_specs=[
            pl.BlockSpec(
                (1, BLK, IN_FEATURES),
                lambda i: (i // (1024 // BLK), i % (1024 // BLK), 0),
            ),
            pl.BlockSpec((IN_FEATURES, NCOL), lambda i: (0, 0)),
            pl.BlockSpec((NCOL,), lambda i: (0,)),
            pl.BlockSpec((1, NCOL), lambda i: (0, 0)),
        ],
        out_specs=[
            pl.BlockSpec((IDR, 128), lambda i: (i, 0)),
            pl.BlockSpec((1, 1), lambda i: (0, 0)),
        ],
        out_shape=[
            jax.ShapeDtypeStruct((IDR * GRID, 128), jnp.int32),
            jax.ShapeDtypeStruct((1, 1), jnp.float32),
        ],
        scratch_shapes=[pltpu.VMEM((1, NCOL), jnp.float32)],
    )(x, w, b_vec, iota_row)


@functools.lru_cache(maxsize=1)
def _make_sc_gather():
    @functools.partial(
        pl.kernel,
        mesh=plsc.VectorSubcoreMesh(core_axis_name="c", subcore_axis_name="s"),
        out_type=jax.ShapeDtypeStruct((ROWS, NUM_CODEBOOKS * CODEWORD_DIM), jnp.float32),
        scratch_types=[
            pltpu.VMEM((2, 128), jnp.int32),
            pltpu.VMEM((256, CODEWORD_DIM), jnp.float32),
            pltpu.SemaphoreType.DMA,
            pltpu.SemaphoreType.DMA,
        ],
    )
    def _sc_gather(table_hbm, idx_hbm, out_hbm, idx_v, rows_v, sem, wsem):
        wid = lax.axis_index("s") * NC + lax.axis_index("c")
        g = wid // W_PER_G
        q = wid % W_PER_G
        n = q // T_PER_CB
        t = q % T_PER_CB
        pltpu.sync_copy(
            idx_hbm.at[pl.ds(IDR * g + (IDR // 2) * n + 2 * t, 2)], idx_v
        )
        row0 = BLK * g + 256 * t
        col = 128 * n
        gathers = [
            pltpu.async_copy(
                table_hbm.at[idx_v.at[j]], rows_v.at[pl.ds(j * 128, 128)], sem
            )
            for j in range(2)
        ]
        writes = []
        for j in range(2):
            gathers[j].wait()
            writes.append(
                pltpu.async_copy(
                    rows_v.at[pl.ds(j * 128, 128)],
                    out_hbm.at[pl.ds(row0 + j * 128, 128), pl.ds(col, 128)],
                    wsem,
                )
            )
        for w in writes:
            w.wait()

    return _sc_gather


_IOTA_ROW = np.arange(NCOL, dtype=np.int32).reshape(1, NCOL)


def kernel(x, codebooks, W, b):
    bsz, nf, _ = x.shape
    ids, perp = _tc_stage(x, W, b, jnp.asarray(_IOTA_ROW))
    table = codebooks.reshape(NCOL, CODEWORD_DIM)
    rows = _make_sc_gather()(table, ids)
    quantized = rows.reshape(bsz, nf, NUM_CODEBOOKS * CODEWORD_DIM)
    return quantized, perp.reshape(())


# speedup vs baseline: 1.0658x; 1.0280x over previous
import functools

import numpy as np

import jax
import jax.numpy as jnp
from jax import lax
from jax.experimental import pallas as pl
from jax.experimental.pallas import tpu as pltpu
from jax.experimental.pallas import tpu_sc as plsc

IN_FEATURES = 512
NUM_CODEBOOKS = 2
NUM_CODEWORDS = 320
NCOL = NUM_CODEBOOKS * NUM_CODEWORDS
CODEWORD_DIM = 128
ROWS = 4 * 1024
BLK = 512
GRID = ROWS // BLK
IDR = NUM_CODEBOOKS * BLK // 128

NC, NS = 2, 16
NW = NC * NS
W_PER_G = NW // GRID
T_PER_CB = BLK // 256


def _tc_body(x_ref, w_ref, b_ref, iota_ref, ids_ref, perp_ref, counts_ref):
    pid = pl.program_id(0)

    @pl.when(pid == 0)
    def _init():
        counts_ref[...] = jnp.zeros_like(counts_ref)

    logits = (
        jnp.dot(x_ref[0], w_ref[...], preferred_element_type=jnp.float32)
        + b_ref[...].reshape(1, NCOL)
    )
    del iota_ref
    iota_b = lax.broadcasted_iota(jnp.int32, (BLK, NCOL), 1)
    big = jnp.int32(2**30)
    neg = jnp.float32(-1e30)
    mask0 = iota_b < NUM_CODEWORDS
    idxs = []
    for n in range(NUM_CODEBOOKS):
        mb = mask0 if n == 0 else jnp.logical_not(mask0)
        lm = jnp.where(mb, logits, neg)
        m = jnp.max(lm, axis=1, keepdims=True)
        cand = jnp.where(lm == m, iota_b, big)
        idxs.append(jnp.min(cand, axis=1, keepdims=True))
    onehot = ((iota_b == idxs[0]) | (iota_b == idxs[1])).astype(jnp.float32)
    counts_ref[...] += lax.dot_general(
        jnp.ones((1, BLK), jnp.float32),
        onehot,
        (((1,), (0,)), ((), ())),
        preferred_element_type=jnp.float32,
    )
    idx_lane = jnp.concatenate(idxs, axis=1).T
    pieces = [
        idx_lane[n : n + 1, 128 * k : 128 * (k + 1)]
        for n in range(NUM_CODEBOOKS)
        for k in range(BLK // 128)
    ]
    ids_ref[...] = jnp.concatenate(pieces, axis=0)

    @pl.when(pid == GRID - 1)
    def _fin():
        p = counts_ref[...] * (1.0 / ROWS)
        plogp = p * jnp.log(p + 1e-7)
        e0 = jnp.sum(plogp[:, :NUM_CODEWORDS])
        e1 = jnp.sum(plogp[:, NUM_CODEWORDS:])
        perp_ref[...] = jnp.broadcast_to(jnp.exp(-e0) + jnp.exp(-e1), (1, 1))


def _tc_stage(x, w, b_vec, iota_row):
    return pl.pallas_call(
        _tc_body,
        grid=(GRID,),
        in_specs=[
            pl.BlockSpec(
                (1, BLK, IN_FEATURES),
                lambda i: (i // (1024 // BLK), i % (1024 // BLK), 0),
            ),
            pl.BlockSpec((IN_FEATURES, NCOL), lambda i: (0, 0)),
            pl.BlockSpec((NCOL,), lambda i: (0,)),
            pl.BlockSpec((1, NCOL), lambda i: (0, 0)),
        ],
        out_specs=[
            pl.BlockSpec((IDR, 128), lambda i: (i, 0)),
            pl.BlockSpec((1, 1), lambda i: (0, 0)),
        ],
        out_shape=[
            jax.ShapeDtypeStruct((IDR * GRID, 128), jnp.int32),
            jax.ShapeDtypeStruct((1, 1), jnp.float32),
        ],
        scratch_shapes=[pltpu.VMEM((1, NCOL), jnp.float32)],
    )(x, w, b_vec, iota_row)


@functools.lru_cache(maxsize=1)
def _make_sc_gather():
    @functools.partial(
        pl.kernel,
        mesh=plsc.VectorSubcoreMesh(core_axis_name="c", subcore_axis_name="s"),
        out_type=jax.ShapeDtypeStruct((ROWS, NUM_CODEBOOKS * CODEWORD_DIM), jnp.float32),
        scratch_types=[
            pltpu.VMEM((2, 128), jnp.int32),
            pltpu.VMEM((256, CODEWORD_DIM), jnp.float32),
            pltpu.SemaphoreType.DMA,
            pltpu.SemaphoreType.DMA,
        ],
    )
    def _sc_gather(table_hbm, idx_hbm, out_hbm, idx_v, rows_v, sem, wsem):
        wid = lax.axis_index("s") * NC + lax.axis_index("c")
        g = wid // W_PER_G
        q = wid % W_PER_G
        n = q // T_PER_CB
        t = q % T_PER_CB
        pltpu.sync_copy(
            idx_hbm.at[pl.ds(IDR * g + (IDR // 2) * n + 2 * t, 2)], idx_v
        )
        row0 = BLK * g + 256 * t
        col = 128 * n
        gathers = [
            pltpu.async_copy(
                table_hbm.at[idx_v.at[j]], rows_v.at[pl.ds(j * 128, 128)], sem
            )
            for j in range(2)
        ]
        writes = []
        for j in range(2):
            gathers[j].wait()
            writes.append(
                pltpu.async_copy(
                    rows_v.at[pl.ds(j * 128, 128)],
                    out_hbm.at[pl.ds(row0 + j * 128, 128), pl.ds(col, 128)],
                    wsem,
                )
            )
        for w in writes:
            w.wait()

    return _sc_gather


_IOTA_ROW = np.arange(NCOL, dtype=np.int32).reshape(1, NCOL)


def kernel(x, codebooks, W, b):
    bsz, nf, _ = x.shape
    ids, perp = _tc_stage(x, W, b, jnp.asarray(_IOTA_ROW))
    table = codebooks.reshape(NCOL, CODEWORD_DIM)
    rows = _make_sc_gather()(table, ids)
    quantized = rows.reshape(bsz, nf, NUM_CODEBOOKS * CODEWORD_DIM)
    return quantized, perp.reshape(())
